# baseline (speedup 1.0000x reference)
import jax
import jax.numpy as jnp
from jax.experimental import pallas as pl
from jax.experimental.pallas import tpu as pltpu


def _proj_kernel(x_ref, w_ref, b_ref, h_ref):
    h_ref[...] = (
        jnp.dot(x_ref[...], w_ref[...], preferred_element_type=jnp.float32)
        + b_ref[...]
    )


def _hop_kernel(adj_ref, agg_ref, d_ref, out_ref):
    a = jnp.dot(adj_ref[...], agg_ref[...], preferred_element_type=jnp.float32)
    out_ref[...] = a * d_ref[...]


def _last_hop_kernel(adj_ref, agg_ref, h_ref, d_ref, out_ref):
    a = jnp.dot(adj_ref[...], agg_ref[...], preferred_element_type=jnp.float32)
    out_ref[...] = jnp.maximum(h_ref[...] - a * d_ref[...], 0.0)


def _fused_kernel(x_ref, w_ref, b_ref, adj_lo_ref, adj_hi_ref, d_ref, out_ref,
                  h_scratch):
    i = pl.program_id(0)

    @pl.when(i == 0)
    def _():
        h_scratch[...] = (
            jnp.dot(x_ref[...], w_ref[...], preferred_element_type=jnp.float32)
            + b_ref[...]
        )

    hm = adj_lo_ref.shape[0]
    h = h_scratch[...]
    a0 = jnp.dot(adj_lo_ref[...], h, preferred_element_type=jnp.float32)
    a1 = jnp.dot(adj_hi_ref[...], h, preferred_element_type=jnp.float32)
    base = i * 2 * hm
    h0 = h_scratch[pl.ds(base, hm), :]
    d0 = d_ref[pl.ds(base, hm), :]
    h1 = h_scratch[pl.ds(base + hm, hm), :]
    d1 = d_ref[pl.ds(base + hm, hm), :]
    out_ref[pl.ds(0, hm), :] = jnp.maximum(h0 - a0 * d0, 0.0)
    out_ref[pl.ds(hm, hm), :] = jnp.maximum(h1 - a1 * d1, 0.0)


def _row_block(n: int) -> int:
    for bm in (400, 200, 1000, 80, 40, 16, 8):
        if n % bm == 0:
            return bm
    return n


def kernel(x, adj_matrix, degree_norm, num_avg, W, b):
    n, feat = x.shape
    hid = W.shape[1]
    b2 = b.reshape(1, hid)
    bm = _row_block(n)
    grid = (n // bm,)
    out_shape = jax.ShapeDtypeStruct((n, hid), jnp.float32)

    def single_hop(_):
        return pl.pallas_call(
            _fused_kernel,
            grid=grid,
            in_specs=[
                pl.BlockSpec((n, feat), lambda i: (0, 0)),
                pl.BlockSpec((feat, hid), lambda i: (0, 0)),
                pl.BlockSpec((1, hid), lambda i: (0, 0)),
                pl.BlockSpec((bm // 2, n), lambda i: (2 * i, 0)),
                pl.BlockSpec((bm // 2, n), lambda i: (2 * i + 1, 0)),
                pl.BlockSpec((n, 1), lambda i: (0, 0)),
            ],
            out_specs=pl.BlockSpec((bm, hid), lambda i: (i, 0)),
            out_shape=out_shape,
            scratch_shapes=[pltpu.VMEM((n, hid), jnp.float32)],
        )(x, W, b2, adj_matrix, adj_matrix, degree_norm)

    def multi_hop(_):
        h = pl.pallas_call(_proj_kernel, out_shape=out_shape)(x, W, b2)

        def hop(agg):
            return pl.pallas_call(
                _hop_kernel,
                grid=grid,
                in_specs=[
                    pl.BlockSpec((bm, n), lambda i: (i, 0)),
                    pl.BlockSpec((n, hid), lambda i: (0, 0)),
                    pl.BlockSpec((bm, 1), lambda i: (i, 0)),
                ],
                out_specs=pl.BlockSpec((bm, hid), lambda i: (i, 0)),
                out_shape=out_shape,
            )(adj_matrix, agg, degree_norm)

        agg = jax.lax.fori_loop(0, num_avg - 1, lambda _, a: hop(a), h)
        return pl.pallas_call(
            _last_hop_kernel,
            grid=grid,
            in_specs=[
                pl.BlockSpec((bm, n), lambda i: (i, 0)),
                pl.BlockSpec((n, hid), lambda i: (0, 0)),
                pl.BlockSpec((bm, hid), lambda i: (i, 0)),
                pl.BlockSpec((bm, 1), lambda i: (i, 0)),
            ],
            out_specs=pl.BlockSpec((bm, hid), lambda i: (i, 0)),
            out_shape=out_shape,
        )(adj_matrix, agg, h, degree_norm)

    return jax.lax.cond(num_avg == 1, single_hop, multi_hop, None)


# speedup vs baseline: 1.0122x; 1.0122x over previous
import jax
import jax.numpy as jnp
from jax.experimental import pallas as pl
from jax.experimental.pallas import tpu as pltpu


def _proj_kernel(x_ref, w_ref, b_ref, h_ref):
    h_ref[...] = (
        jnp.dot(x_ref[...], w_ref[...], preferred_element_type=jnp.float32)
        + b_ref[...]
    )


def _hop_kernel(adj_ref, agg_ref, d_ref, out_ref):
    a = jnp.dot(adj_ref[...], agg_ref[...], preferred_element_type=jnp.float32)
    out_ref[...] = a * d_ref[...]


def _last_hop_kernel(adj_ref, agg_ref, h_ref, d_ref, out_ref):
    a = jnp.dot(adj_ref[...], agg_ref[...], preferred_element_type=jnp.float32)
    out_ref[...] = jnp.maximum(h_ref[...] - a * d_ref[...], 0.0)


def _fused_kernel(x_ref, w_ref, b_ref, adj_ref, d_ref, out_ref, h_scratch):
    i = pl.program_id(0)

    @pl.when(i == 0)
    def _():
        h_scratch[...] = (
            jnp.dot(x_ref[...], w_ref[...], preferred_element_type=jnp.float32)
            + b_ref[...]
        )

    bm = out_ref.shape[0]
    a = jnp.dot(adj_ref[...], h_scratch[...], preferred_element_type=jnp.float32)
    h_rows = h_scratch[pl.ds(i * bm, bm), :]
    d_rows = d_ref[pl.ds(i * bm, bm), :]
    out_ref[...] = jnp.maximum(h_rows - a * d_rows, 0.0)


def _row_block(n: int) -> int:
    for bm in (200, 400, 80, 40, 16, 8):
        if n % bm == 0:
            return bm
    return n


def kernel(x, adj_matrix, degree_norm, num_avg, W, b):
    n, feat = x.shape
    hid = W.shape[1]
    b2 = b.reshape(1, hid)
    bm = _row_block(n)
    grid = (n // bm,)
    out_shape = jax.ShapeDtypeStruct((n, hid), jnp.float32)

    def single_hop(_):
        return pl.pallas_call(
            _fused_kernel,
            grid=grid,
            in_specs=[
                pl.BlockSpec((n, feat), lambda i: (0, 0)),
                pl.BlockSpec((feat, hid), lambda i: (0, 0)),
                pl.BlockSpec((1, hid), lambda i: (0, 0)),
                pl.BlockSpec((bm, n), lambda i: (i, 0)),
                pl.BlockSpec((n, 1), lambda i: (0, 0)),
            ],
            out_specs=pl.BlockSpec((bm, hid), lambda i: (i, 0)),
            out_shape=out_shape,
            scratch_shapes=[pltpu.VMEM((n, hid), jnp.float32)],
        )(x, W, b2, adj_matrix, degree_norm)

    def multi_hop(_):
        h = pl.pallas_call(_proj_kernel, out_shape=out_shape)(x, W, b2)

        def hop(agg):
            return pl.pallas_call(
                _hop_kernel,
                grid=grid,
                in_specs=[
                    pl.BlockSpec((bm, n), lambda i: (i, 0)),
                    pl.BlockSpec((n, hid), lambda i: (0, 0)),
                    pl.BlockSpec((bm, 1), lambda i: (i, 0)),
                ],
                out_specs=pl.BlockSpec((bm, hid), lambda i: (i, 0)),
                out_shape=out_shape,
            )(adj_matrix, agg, degree_norm)

        agg = jax.lax.fori_loop(0, num_avg - 1, lambda _, a: hop(a), h)
        return pl.pallas_call(
            _last_hop_kernel,
            grid=grid,
            in_specs=[
                pl.BlockSpec((bm, n), lambda i: (i, 0)),
                pl.BlockSpec((n, hid), lambda i: (0, 0)),
                pl.BlockSpec((bm, hid), lambda i: (i, 0)),
                pl.BlockSpec((bm, 1), lambda i: (i, 0)),
            ],
            out_specs=pl.BlockSpec((bm, hid), lambda i: (i, 0)),
            out_shape=out_shape,
        )(adj_matrix, agg, h, degree_norm)

    return jax.lax.cond(num_avg == 1, single_hop, multi_hop, None)


# speedup vs baseline: 1.0161x; 1.0038x over previous
import jax
import jax.numpy as jnp
from jax.experimental import pallas as pl
from jax.experimental.pallas import tpu as pltpu


def _proj_kernel(x_ref, w_ref, b_ref, h_ref):
    h_ref[...] = (
        jnp.dot(x_ref[...], w_ref[...], preferred_element_type=jnp.float32)
        + b_ref[...]
    )


def _hop_kernel(adj_ref, agg_ref, d_ref, out_ref):
    a = jnp.dot(adj_ref[...], agg_ref[...], preferred_element_type=jnp.float32)
    out_ref[...] = a * d_ref[...]


def _last_hop_kernel(adj_ref, agg_ref, h_ref, d_ref, out_ref):
    a = jnp.dot(adj_ref[...], agg_ref[...], preferred_element_type=jnp.float32)
    out_ref[...] = jnp.maximum(h_ref[...] - a * d_ref[...], 0.0)


def _fused_kernel(x_ref, w_ref, b_ref, adj_ref, d_ref, out_ref, h_scratch):
    i = pl.program_id(0)

    @pl.when(i == 0)
    def _():
        h_scratch[...] = (
            jnp.dot(x_ref[...], w_ref[...], preferred_element_type=jnp.float32)
            + b_ref[...]
        )

    bm = out_ref.shape[0]
    a = jnp.dot(adj_ref[...], h_scratch[...], preferred_element_type=jnp.float32)
    h_rows = h_scratch[pl.ds(i * bm, bm), :]
    d_rows = d_ref[pl.ds(i * bm, bm), :]
    out_ref[...] = jnp.maximum(h_rows - a * d_rows, 0.0)


def _row_block(n: int) -> int:
    for bm in (400, 200, 80, 40, 16, 8):
        if n % bm == 0:
            return bm
    return n


def kernel(x, adj_matrix, degree_norm, num_avg, W, b):
    n, feat = x.shape
    hid = W.shape[1]
    b2 = b.reshape(1, hid)
    bm = _row_block(n)
    grid = (n // bm,)
    out_shape = jax.ShapeDtypeStruct((n, hid), jnp.float32)

    def single_hop(_):
        return pl.pallas_call(
            _fused_kernel,
            grid=grid,
            in_specs=[
                pl.BlockSpec((n, feat), lambda i: (0, 0)),
                pl.BlockSpec((feat, hid), lambda i: (0, 0)),
                pl.BlockSpec((1, hid), lambda i: (0, 0)),
                pl.BlockSpec((bm, n), lambda i: (i, 0)),
                pl.BlockSpec((n, 1), lambda i: (0, 0)),
            ],
            out_specs=pl.BlockSpec((bm, hid), lambda i: (i, 0)),
            out_shape=out_shape,
            scratch_shapes=[pltpu.VMEM((n, hid), jnp.float32)],
        )(x, W, b2, adj_matrix, degree_norm)

    def multi_hop(_):
        h = pl.pallas_call(_proj_kernel, out_shape=out_shape)(x, W, b2)

        def hop(agg):
            return pl.pallas_call(
                _hop_kernel,
                grid=grid,
                in_specs=[
                    pl.BlockSpec((bm, n), lambda i: (i, 0)),
                    pl.BlockSpec((n, hid), lambda i: (0, 0)),
                    pl.BlockSpec((bm, 1), lambda i: (i, 0)),
                ],
                out_specs=pl.BlockSpec((bm, hid), lambda i: (i, 0)),
                out_shape=out_shape,
            )(adj_matrix, agg, degree_norm)

        agg = jax.lax.fori_loop(0, num_avg - 1, lambda _, a: hop(a), h)
        return pl.pallas_call(
            _last_hop_kernel,
            grid=grid,
            in_specs=[
                pl.BlockSpec((bm, n), lambda i: (i, 0)),
                pl.BlockSpec((n, hid), lambda i: (0, 0)),
                pl.BlockSpec((bm, hid), lambda i: (i, 0)),
                pl.BlockSpec((bm, 1), lambda i: (i, 0)),
            ],
            out_specs=pl.BlockSpec((bm, hid), lambda i: (i, 0)),
            out_shape=out_shape,
        )(adj_matrix, agg, h, degree_norm)

    return jax.lax.cond(num_avg == 1, single_hop, multi_hop, None)
